# baseline (speedup 1.0000x reference)
import functools

import jax
import jax.numpy as jnp
from jax import lax
from jax.experimental import pallas as pl
from jax.experimental.pallas import tpu as pltpu
from jax.experimental.pallas import tpu_sc as plsc

QS = 0.01


def _knn_kernel(coords_ref, xyz_ref, feats_ref, w_ref, idx_ref, wt_ref, f_ref,
                *, nt, m):
    b = pl.program_id(0)

    @pl.when(pl.program_id(1) == 0)
    def _():
        fm = jax.lax.dot(feats_ref[0], w_ref[...],
                         preferred_element_type=jnp.float32)
        f_ref[0] = jnp.concatenate([fm, jnp.zeros_like(fm)], axis=1)

    src = coords_ref[0].astype(jnp.float32) * QS
    q = xyz_ref[0]
    d2 = ((q[:, 0:1] - src[0:1, :]) ** 2
          + (q[:, 1:2] - src[1:2, :]) ** 2
          + (q[:, 2:3] - src[2:3, :]) ** 2)

    iota = jax.lax.broadcasted_iota(jnp.int32, (nt, m), 1).astype(jnp.float32)
    inf = jnp.float32(jnp.inf)
    mf = jnp.float32(m)

    d = d2
    ixs = []
    rs = []
    rsum = jnp.zeros((nt, 1), jnp.float32)
    for k in range(3):
        mn = jnp.min(d, axis=1, keepdims=True)
        ix = jnp.min(jnp.where(d == mn, iota, mf), axis=1, keepdims=True)
        r = 1.0 / (mn + 1e-8)
        rsum = rsum + r
        ixs.append(ix)
        rs.append(r)
        if k < 2:
            d = jnp.where(iota == ix, inf, d)
    inv_norm = 1.0 / rsum

    base = b * m
    idx_ref[0] = jnp.concatenate(
        [ix.astype(jnp.int32) + base for ix in ixs], axis=1)
    out_dim = wt_ref.shape[-1]
    for k in range(3):
        wt_ref[0, k] = jnp.broadcast_to(rs[k] * inv_norm,
                                        (ixs[0].shape[0], out_dim))


CHUNK = 64
QPW = 128
NCHUNK = QPW // CHUNK


def _sc_interp_kernel(f_hbm, idx_hbm, w_hbm, out_hbm,
                      i0, i1, i2,
                      r0a, r1a, r2a, r0b, r1b, r2b,
                      w0a, w1a, w2a, w0b, w1b, w2b,
                      outv, gsem, wsem):
    nc = 2
    wid = lax.axis_index("s") * nc + lax.axis_index("c")
    qoff = wid * QPW
    n = 4096
    rows = ((r0a, r1a, r2a), (r0b, r1b, r2b))
    wvs = ((w0a, w1a, w2a), (w0b, w1b, w2b))
    idxs = (i0, i1, i2)
    for k in range(3):
        pltpu.sync_copy(idx_hbm.at[pl.ds(k * n + qoff, QPW)], idxs[k])

    def fire(h, buf):
        handles = []
        for k in range(3):
            handles.append(pltpu.async_copy(
                f_hbm.at[idxs[k].at[pl.ds(h * CHUNK, CHUNK)]],
                rows[buf][k], gsem))
            handles.append(pltpu.async_copy(
                w_hbm.at[pl.ds(k * n + qoff + h * CHUNK, CHUNK)],
                wvs[buf][k], wsem))
        return handles

    pend = fire(0, 0)
    for h in range(NCHUNK):
        cur = h % 2
        for hd in pend:
            hd.wait()
        if h + 1 < NCHUNK:
            pend = fire(h + 1, 1 - cur)
        ra, rb, rc = rows[cur]
        wa, wb, wc = wvs[cur]

        def body(q, carry, *, ra=ra, rb=rb, rc=rc, wa=wa, wb=wb, wc=wc):
            for cg in range(4):
                sl = pl.ds(cg * 16, 16)
                acc = (ra[q, sl] * wa[q, sl]
                       + rb[q, sl] * wb[q, sl]
                       + rc[q, sl] * wc[q, sl])
                outv[q, sl] = acc
            return carry

        jax.lax.fori_loop(0, CHUNK, body, 0)
        pltpu.sync_copy(outv, out_hbm.at[pl.ds(qoff + h * CHUNK, CHUNK)])


def _ln_kernel(interp_ref, g_ref, b_ref, out_ref):
    x = interp_ref[0]
    mu = jnp.mean(x, axis=1, keepdims=True)
    xc = x - mu
    var = jnp.mean(xc * xc, axis=1, keepdims=True)
    xn = xc / jnp.sqrt(var + 1e-6) * g_ref[0] + b_ref[0]
    inv_sqrt2 = jnp.float32(0.7071067811865476)
    act = 0.5 * xn * (1.0 + jax.lax.erf(xn * inv_sqrt2))
    out_ref[0] = act.T


def kernel(feats, coords, xyz_t, W, ln_gamma, ln_beta):
    B, M, INP = feats.shape
    _, N, _ = xyz_t.shape
    OUT = W.shape[1]
    NT = 1024

    coords_t = jnp.transpose(coords, (0, 2, 1))
    gamma2 = ln_gamma.reshape(1, OUT)
    beta2 = ln_beta.reshape(1, OUT)

    knn_call = pl.pallas_call(
        functools.partial(_knn_kernel, nt=NT, m=M),
        grid=(1, N // NT),
        in_specs=[
            pl.BlockSpec((1, 3, M), lambda b, n: (b, 0, 0)),
            pl.BlockSpec((1, NT, 3), lambda b, n: (b, n, 0)),
            pl.BlockSpec((1, M, INP), lambda b, n: (b, 0, 0)),
            pl.BlockSpec((INP, OUT), lambda b, n: (0, 0)),
        ],
        out_specs=[
            pl.BlockSpec((1, NT, 3), lambda b, n: (b, n, 0)),
            pl.BlockSpec((1, 3, NT, OUT), lambda b, n: (b, 0, n, 0)),
            pl.BlockSpec((1, M, 2 * OUT), lambda b, n: (b, 0, 0)),
        ],
        out_shape=[
            jax.ShapeDtypeStruct((1, N, 3), jnp.int32),
            jax.ShapeDtypeStruct((1, 3, N, OUT), jnp.float32),
            jax.ShapeDtypeStruct((1, M, 2 * OUT), jnp.float32),
        ],
        compiler_params=pltpu.CompilerParams(
            dimension_semantics=("arbitrary", "arbitrary"),
        ),
    )

    mesh = plsc.VectorSubcoreMesh(core_axis_name="c", subcore_axis_name="s")
    sc_call = functools.partial(
        pl.kernel, mesh=mesh,
        out_type=jax.ShapeDtypeStruct((N, OUT), jnp.float32),
        scratch_types=(
            [pltpu.VMEM((QPW,), jnp.int32)] * 3
            + [pltpu.VMEM((CHUNK, 2 * OUT), jnp.float32)] * 6
            + [pltpu.VMEM((CHUNK, OUT), jnp.float32)] * 6
            + [pltpu.VMEM((CHUNK, OUT), jnp.float32)]
            + [pltpu.SemaphoreType.DMA, pltpu.SemaphoreType.DMA]
        ),
    )(_sc_interp_kernel)

    knns = [knn_call(coords_t[b:b + 1], xyz_t[b:b + 1], feats[b:b + 1], W)
            for b in range(B)]
    interps = []
    for b in range(B):
        knn_i, knn_w, f = knns[b]
        f_flat = f.reshape(M, 2 * OUT)
        idx_t = jnp.transpose(knn_i[0], (1, 0)).reshape(-1)
        w_t = knn_w.reshape(3 * N, OUT)
        interps.append(sc_call(f_flat, idx_t, w_t))
    interp = jnp.stack(interps, axis=0)

    NT3 = 2048
    out = pl.pallas_call(
        _ln_kernel,
        grid=(B, N // NT3),
        in_specs=[
            pl.BlockSpec((1, NT3, OUT), lambda b, n: (b, n, 0)),
            pl.BlockSpec((1, OUT), lambda b, n: (0, 0)),
            pl.BlockSpec((1, OUT), lambda b, n: (0, 0)),
        ],
        out_specs=pl.BlockSpec((1, OUT, NT3), lambda b, n: (b, 0, n)),
        out_shape=jax.ShapeDtypeStruct((B, OUT, N), jnp.float32),
    )(interp, gamma2, beta2)
    return out


# speedup vs baseline: 1.0490x; 1.0490x over previous
import functools

import jax
import jax.numpy as jnp
from jax import lax
from jax.experimental import pallas as pl
from jax.experimental.pallas import tpu as pltpu
from jax.experimental.pallas import tpu_sc as plsc

QS = 0.01


def _knn_kernel(coords_ref, xyz_ref, feats_ref, w_ref, idx_ref, wt_ref, f_ref,
                *, nt, m):
    b = pl.program_id(0)

    @pl.when(pl.program_id(1) == 0)
    def _():
        fm = jax.lax.dot(feats_ref[0], w_ref[...],
                         preferred_element_type=jnp.float32)
        f_ref[0] = jnp.concatenate([fm, jnp.zeros_like(fm)], axis=1)

    src = coords_ref[0].astype(jnp.float32) * QS
    q = xyz_ref[0]
    d2 = ((q[:, 0:1] - src[0:1, :]) ** 2
          + (q[:, 1:2] - src[1:2, :]) ** 2
          + (q[:, 2:3] - src[2:3, :]) ** 2)

    iota = jax.lax.broadcasted_iota(jnp.int32, (nt, m), 1).astype(jnp.float32)
    inf = jnp.float32(jnp.inf)
    mf = jnp.float32(m)

    d = d2
    ixs = []
    rs = []
    rsum = jnp.zeros((nt, 1), jnp.float32)
    for k in range(3):
        mn = jnp.min(d, axis=1, keepdims=True)
        ix = jnp.min(jnp.where(d == mn, iota, mf), axis=1, keepdims=True)
        r = 1.0 / (mn + 1e-8)
        rsum = rsum + r
        ixs.append(ix)
        rs.append(r)
        if k < 2:
            d = jnp.where(iota == ix, inf, d)
    inv_norm = 1.0 / rsum

    base = b * m
    idx_ref[0] = jnp.concatenate(
        [ix.astype(jnp.int32) + base for ix in ixs], axis=1)
    out_dim = wt_ref.shape[-1]
    for k in range(3):
        wt_ref[0, k] = jnp.broadcast_to(rs[k] * inv_norm,
                                        (ixs[0].shape[0], out_dim))


CHUNK = 64
QPW = 256
NCHUNK = QPW // CHUNK


def _sc_interp_kernel(f_hbm, idx_hbm, w_hbm, out_hbm,
                      i0, i1, i2,
                      r0a, r1a, r2a, r0b, r1b, r2b,
                      w0a, w1a, w2a, w0b, w1b, w2b,
                      outv, gsem, wsem):
    nc = 2
    wid = lax.axis_index("s") * nc + lax.axis_index("c")
    b = wid // 16
    qoff = (wid % 16) * QPW
    n = 4096
    rows = ((r0a, r1a, r2a), (r0b, r1b, r2b))
    wvs = ((w0a, w1a, w2a), (w0b, w1b, w2b))
    idxs = (i0, i1, i2)
    for k in range(3):
        pltpu.sync_copy(idx_hbm.at[pl.ds((b * 3 + k) * n + qoff, QPW)],
                        idxs[k])

    def fire(h, buf):
        handles = []
        for k in range(3):
            handles.append(pltpu.async_copy(
                f_hbm.at[idxs[k].at[pl.ds(h * CHUNK, CHUNK)]],
                rows[buf][k], gsem))
            handles.append(pltpu.async_copy(
                w_hbm.at[pl.ds((b * 3 + k) * n + qoff + h * CHUNK, CHUNK)],
                wvs[buf][k], wsem))
        return handles

    pend = fire(0, 0)
    for h in range(NCHUNK):
        cur = h % 2
        for hd in pend:
            hd.wait()
        if h + 1 < NCHUNK:
            pend = fire(h + 1, 1 - cur)
        ra, rb, rc = rows[cur]
        wa, wb, wc = wvs[cur]

        def body(q, carry, *, ra=ra, rb=rb, rc=rc, wa=wa, wb=wb, wc=wc):
            for cg in range(4):
                sl = pl.ds(cg * 16, 16)
                acc = (ra[q, sl] * wa[q, sl]
                       + rb[q, sl] * wb[q, sl]
                       + rc[q, sl] * wc[q, sl])
                outv[q, sl] = acc
            return carry

        jax.lax.fori_loop(0, CHUNK, body, 0)
        pltpu.sync_copy(outv,
                        out_hbm.at[pl.ds(wid * QPW + h * CHUNK, CHUNK)])


def _ln_kernel(interp_ref, g_ref, b_ref, out_ref):
    x = interp_ref[0]
    mu = jnp.mean(x, axis=1, keepdims=True)
    xc = x - mu
    var = jnp.mean(xc * xc, axis=1, keepdims=True)
    xn = xc / jnp.sqrt(var + 1e-6) * g_ref[0] + b_ref[0]
    inv_sqrt2 = jnp.float32(0.7071067811865476)
    act = 0.5 * xn * (1.0 + jax.lax.erf(xn * inv_sqrt2))
    out_ref[0] = act.T


def kernel(feats, coords, xyz_t, W, ln_gamma, ln_beta):
    B, M, INP = feats.shape
    _, N, _ = xyz_t.shape
    OUT = W.shape[1]
    NT = 1024

    coords_t = jnp.transpose(coords, (0, 2, 1))
    gamma2 = ln_gamma.reshape(1, OUT)
    beta2 = ln_beta.reshape(1, OUT)

    knn_i, knn_w, f = pl.pallas_call(
        functools.partial(_knn_kernel, nt=NT, m=M),
        grid=(B, N // NT),
        in_specs=[
            pl.BlockSpec((1, 3, M), lambda b, n: (b, 0, 0)),
            pl.BlockSpec((1, NT, 3), lambda b, n: (b, n, 0)),
            pl.BlockSpec((1, M, INP), lambda b, n: (b, 0, 0)),
            pl.BlockSpec((INP, OUT), lambda b, n: (0, 0)),
        ],
        out_specs=[
            pl.BlockSpec((1, NT, 3), lambda b, n: (b, n, 0)),
            pl.BlockSpec((1, 3, NT, OUT), lambda b, n: (b, 0, n, 0)),
            pl.BlockSpec((1, M, 2 * OUT), lambda b, n: (b, 0, 0)),
        ],
        out_shape=[
            jax.ShapeDtypeStruct((B, N, 3), jnp.int32),
            jax.ShapeDtypeStruct((B, 3, N, OUT), jnp.float32),
            jax.ShapeDtypeStruct((B, M, 2 * OUT), jnp.float32),
        ],
        compiler_params=pltpu.CompilerParams(
            dimension_semantics=("arbitrary", "arbitrary"),
        ),
    )(coords_t, xyz_t, feats, W)

    f_flat = f.reshape(B * M, 2 * OUT)
    idx_t = jnp.transpose(knn_i, (0, 2, 1)).reshape(-1)
    w_t = knn_w.reshape(B * 3 * N, OUT)

    mesh = plsc.VectorSubcoreMesh(core_axis_name="c", subcore_axis_name="s")
    sc_call = functools.partial(
        pl.kernel, mesh=mesh,
        out_type=jax.ShapeDtypeStruct((B * N, OUT), jnp.float32),
        scratch_types=(
            [pltpu.VMEM((QPW,), jnp.int32)] * 3
            + [pltpu.VMEM((CHUNK, 2 * OUT), jnp.float32)] * 6
            + [pltpu.VMEM((CHUNK, OUT), jnp.float32)] * 6
            + [pltpu.VMEM((CHUNK, OUT), jnp.float32)]
            + [pltpu.SemaphoreType.DMA, pltpu.SemaphoreType.DMA]
        ),
    )(_sc_interp_kernel)
    interp = sc_call(f_flat, idx_t, w_t).reshape(B, N, OUT)

    NT3 = 2048
    out = pl.pallas_call(
        _ln_kernel,
        grid=(B, N // NT3),
        in_specs=[
            pl.BlockSpec((1, NT3, OUT), lambda b, n: (b, n, 0)),
            pl.BlockSpec((1, OUT), lambda b, n: (0, 0)),
            pl.BlockSpec((1, OUT), lambda b, n: (0, 0)),
        ],
        out_specs=pl.BlockSpec((1, OUT, NT3), lambda b, n: (b, 0, n)),
        out_shape=jax.ShapeDtypeStruct((B, OUT, N), jnp.float32),
    )(interp, gamma2, beta2)
    return out
